# baseline (speedup 1.0000x reference)
import functools

import jax
import jax.numpy as jnp
from jax import lax
from jax.experimental import pallas as pl
from jax.experimental.pallas import tpu as pltpu
from jax.experimental.pallas import tpu_sc as plsc

N = 10000
E = 320000
D = 128
HD = D // 2
NP = 10112
CH = 128
NT = 16
ECT = E // NT
NCHT = 160
HALF = NCHT // 2
PADT = NCHT * CH - ECT
NBUF = 8
RPT = NP // 16
_ROW_CHUNKS = (128, 128, 128, 128, 120)
BLK = NP // 4
FBLK = 2000

_mesh = plsc.VectorSubcoreMesh(core_axis_name="c", subcore_axis_name="s")


@functools.partial(
    pl.kernel,
    out_type=jax.ShapeDtypeStruct((2 * NP,), jnp.float32),
    mesh=_mesh,
    scratch_types=[
        pltpu.VMEM((HALF, CH), jnp.int32),
        pltpu.VMEM((CH,), jnp.float32),
        pltpu.VMEM((RPT,), jnp.float32),
        pltpu.VMEM_SHARED((NP,), jnp.float32),
    ],
)
def _degree_kernel(src_hbm, ones_hbm, zcol_hbm, dego_hbm,
                   idxs_v, ones_v, zb_v, dego_sh):
    c = lax.axis_index("c")
    s = lax.axis_index("s")
    r0 = s * RPT
    pltpu.sync_copy(src_hbm.at[s, pl.ds(c * HALF, HALF)], idxs_v)
    pltpu.sync_copy(ones_hbm, ones_v)
    pltpu.sync_copy(zcol_hbm.at[pl.ds(r0, RPT)], zb_v)
    pltpu.sync_copy(zb_v, dego_sh.at[pl.ds(r0, RPT)])
    plsc.subcore_barrier()

    def body(g, carry):
        pltpu.sync_copy(ones_v, dego_sh.at[idxs_v.at[g]], add=True)
        return carry

    lax.fori_loop(0, HALF, body, 0)
    plsc.subcore_barrier()
    pltpu.sync_copy(dego_sh.at[pl.ds(r0, RPT)], zb_v)
    pltpu.sync_copy(zb_v, dego_hbm.at[pl.ds(c * NP + r0, RPT)])


@functools.partial(
    pl.kernel,
    out_type=(
        jax.ShapeDtypeStruct((2, NP, HD), jnp.float32),
        jax.ShapeDtypeStruct((2 * NP,), jnp.float32),
    ),
    mesh=_mesh,
    compiler_params=pltpu.CompilerParams(use_tc_tiling_on_sc=False),
    scratch_types=[
        pltpu.VMEM((HALF, CH), jnp.int32),
        pltpu.VMEM((HALF, CH), jnp.int32),
        pltpu.VMEM((NBUF, CH, HD), jnp.float32),
        pltpu.VMEM((CH,), jnp.float32),
        pltpu.VMEM((RPT,), jnp.float32),
        pltpu.VMEM_SHARED((NP, HD), jnp.float32),
        pltpu.VMEM_SHARED((NP,), jnp.float32),
        pltpu.SemaphoreType.DMA,
        pltpu.SemaphoreType.DMA,
        pltpu.SemaphoreType.DMA,
        pltpu.SemaphoreType.DMA,
        pltpu.SemaphoreType.DMA,
        pltpu.SemaphoreType.DMA,
        pltpu.SemaphoreType.DMA,
        pltpu.SemaphoreType.DMA,
        pltpu.SemaphoreType.DMA,
    ],
)
def _aggregate_kernel(h2_hbm, src_hbm, dst_hbm, ones_hbm, zrows_hbm, zcol_hbm,
                      agg_hbm, degi_hbm,
                      idxs_v, idxd_v, buf_v, ones_v, zb_v, agg_sh, degi_sh,
                      *sems):
    gsems = sems[:NBUF]
    osem = sems[NBUF]
    c = lax.axis_index("c")
    s = lax.axis_index("s")
    r0 = s * RPT
    pltpu.sync_copy(ones_hbm, ones_v)
    for j, rows in enumerate(_ROW_CHUNKS):
        pltpu.sync_copy(zrows_hbm.at[pl.ds(r0 + 128 * j, rows)],
                        buf_v.at[0, pl.ds(0, rows)])
        pltpu.sync_copy(buf_v.at[0, pl.ds(0, rows)],
                        agg_sh.at[pl.ds(r0 + 128 * j, rows)])
    pltpu.sync_copy(zcol_hbm.at[pl.ds(r0, RPT)], zb_v)
    pltpu.sync_copy(zb_v, degi_sh.at[pl.ds(r0, RPT)])
    plsc.subcore_barrier()

    def gstart(g, slot):
        pltpu.async_copy(h2_hbm.at[c].at[idxs_v.at[g]], buf_v.at[slot],
                         gsems[slot])

    def gwait(slot):
        pltpu.make_async_copy(h2_hbm.at[c].at[idxs_v.at[0]], buf_v.at[slot],
                              gsems[slot]).wait()

    for p in range(2):
        pltpu.sync_copy(src_hbm.at[s, pl.ds(p * HALF, HALF)], idxs_v)
        pltpu.sync_copy(dst_hbm.at[s, pl.ds(p * HALF, HALF)], idxd_v)
        count_here = jnp.equal(c, p)

        def scat(g, slot):
            pltpu.sync_copy(buf_v.at[slot], agg_sh.at[idxd_v.at[g]], add=True)

            @pl.when(count_here)
            def _():
                pltpu.async_copy(ones_v, degi_sh.at[idxd_v.at[g]], osem,
                                 add=True)

        for b in range(NBUF):
            gstart(b, b)

        def body(i, carry):
            g0 = NBUF * i
            for b in range(NBUF):
                gwait(b)
                scat(g0 + b, b)
                gstart(g0 + b + NBUF, b)
            return carry

        lax.fori_loop(0, (HALF - NBUF) // NBUF, body, 0)
        for b in range(NBUF):
            gwait(b)
            scat(HALF - NBUF + b, b)

        @pl.when(count_here)
        def _():
            def drain(i, carry):
                pltpu.make_async_copy(ones_v, degi_sh.at[idxd_v.at[0]],
                                      osem).wait()
                return carry
            lax.fori_loop(0, HALF, drain, 0)

    plsc.subcore_barrier()
    for j, rows in enumerate(_ROW_CHUNKS):
        pltpu.sync_copy(agg_sh.at[pl.ds(r0 + 128 * j, rows)],
                        buf_v.at[0, pl.ds(0, rows)])
        pltpu.sync_copy(buf_v.at[0, pl.ds(0, rows)],
                        agg_hbm.at[c, pl.ds(r0 + 128 * j, rows)])
    pltpu.sync_copy(degi_sh.at[pl.ds(r0, RPT)], zb_v)
    pltpu.sync_copy(zb_v, degi_hbm.at[pl.ds(c * NP + r0, RPT)])


def _scale_body(x_ref, deg_ref, h2_ref):
    i = pl.program_id(0)
    d = deg_ref[0] + deg_ref[1]
    h = x_ref[...] * lax.rsqrt(jnp.maximum(d, 1.0))
    rows = i * BLK + lax.broadcasted_iota(jnp.int32, (BLK, 1), 0)
    h = jnp.where(rows < N, h, 0.0)
    h2_ref[0] = h[:, :HD]
    h2_ref[1] = h[:, HD:]


def _scale_rows(x, degp):
    return pl.pallas_call(
        _scale_body,
        grid=(NP // BLK,),
        in_specs=[
            pl.BlockSpec((BLK, D), lambda i: (i, 0)),
            pl.BlockSpec((2, BLK, 1), lambda i: (0, i, 0)),
        ],
        out_specs=pl.BlockSpec((2, BLK, HD), lambda i: (0, i, 0)),
        out_shape=jax.ShapeDtypeStruct((2, NP, HD), jnp.float32),
    )(x, degp)


def _finish_body(agg_ref, deg_ref, w_ref, b_ref, y_ref):
    d = deg_ref[0] + deg_ref[1]
    n = lax.rsqrt(jnp.maximum(d, 1.0))
    w = w_ref[...]
    y_ref[...] = (jnp.dot(agg_ref[0] * n, w[:HD, :],
                          preferred_element_type=jnp.float32)
                  + jnp.dot(agg_ref[1] * n, w[HD:, :],
                            preferred_element_type=jnp.float32)
                  + b_ref[...])


def _finish(aggp, degp, W, b2):
    return pl.pallas_call(
        _finish_body,
        grid=(N // FBLK,),
        in_specs=[
            pl.BlockSpec((2, FBLK, HD), lambda i: (0, i, 0)),
            pl.BlockSpec((2, FBLK, 1), lambda i: (0, i, 0)),
            pl.BlockSpec((D, D), lambda i: (0, 0)),
            pl.BlockSpec((1, D), lambda i: (0, 0)),
        ],
        out_specs=pl.BlockSpec((FBLK, D), lambda i: (i, 0)),
        out_shape=jax.ShapeDtypeStruct((N, D), jnp.float32),
    )(aggp, degp, W, b2)


def kernel(x, edge_index, W, b):
    src = edge_index[0]
    dst = edge_index[1]

    pad = (N + jnp.arange(NT, dtype=jnp.int32))[:, None]
    pad = jnp.broadcast_to(pad, (NT, PADT))
    src3 = jnp.concatenate([src.reshape(NT, ECT), pad], 1).reshape(NT, NCHT, CH)
    dst3 = jnp.concatenate([dst.reshape(NT, ECT), pad], 1).reshape(NT, NCHT, CH)

    ones_col = jnp.ones((CH,), jnp.float32)
    zcol = jnp.zeros((NP,), jnp.float32)
    zrows = jnp.zeros((NP, HD), jnp.float32)

    deg_out_f = _degree_kernel(src3, ones_col, zcol)
    h2 = _scale_rows(x, deg_out_f.reshape(2, NP, 1))
    aggp, deg_in_f = _aggregate_kernel(h2, src3, dst3, ones_col, zrows, zcol)
    return _finish(aggp, deg_in_f.reshape(2, NP, 1), W, b.reshape(1, D))


# speedup vs baseline: 1.0091x; 1.0091x over previous
import functools

import jax
import jax.numpy as jnp
from jax import lax
from jax.experimental import pallas as pl
from jax.experimental.pallas import tpu as pltpu
from jax.experimental.pallas import tpu_sc as plsc

N = 10000
E = 320000
D = 128
HD = D // 2
NP = 10112
CH = 128
NT = 16
ECT = E // NT
NCHT = 160
HALF = NCHT // 2
PADT = NCHT * CH - ECT
NBUF = 8
RPT = NP // 16
_ROW_CHUNKS = (128, 128, 128, 128, 120)
BLK = NP // 4
FBLK = 2000

_mesh = plsc.VectorSubcoreMesh(core_axis_name="c", subcore_axis_name="s")


@functools.partial(
    pl.kernel,
    out_type=jax.ShapeDtypeStruct((2 * NP,), jnp.float32),
    mesh=_mesh,
    compiler_params=pltpu.CompilerParams(use_tc_tiling_on_sc=False),
    scratch_types=[
        pltpu.VMEM((HALF, CH), jnp.int32),
        pltpu.VMEM((CH,), jnp.float32),
        pltpu.VMEM((RPT,), jnp.float32),
        pltpu.VMEM_SHARED((NP,), jnp.float32),
    ],
)
def _degree_kernel(src_hbm, ones_hbm, zcol_hbm, dego_hbm,
                   idxs_v, ones_v, zb_v, dego_sh):
    c = lax.axis_index("c")
    s = lax.axis_index("s")
    r0 = s * RPT
    pltpu.sync_copy(src_hbm.at[s, pl.ds(c * HALF, HALF)], idxs_v)
    pltpu.sync_copy(ones_hbm, ones_v)
    pltpu.sync_copy(zcol_hbm.at[pl.ds(r0, RPT)], zb_v)
    pltpu.sync_copy(zb_v, dego_sh.at[pl.ds(r0, RPT)])
    plsc.subcore_barrier()

    def body(g, carry):
        pltpu.sync_copy(ones_v, dego_sh.at[idxs_v.at[g]], add=True)
        return carry

    lax.fori_loop(0, HALF, body, 0)
    plsc.subcore_barrier()
    pltpu.sync_copy(dego_sh.at[pl.ds(r0, RPT)], zb_v)
    pltpu.sync_copy(zb_v, dego_hbm.at[pl.ds(c * NP + r0, RPT)])


@functools.partial(
    pl.kernel,
    out_type=(
        jax.ShapeDtypeStruct((2, NP, HD), jnp.float32),
        jax.ShapeDtypeStruct((2 * NP,), jnp.float32),
    ),
    mesh=_mesh,
    compiler_params=pltpu.CompilerParams(use_tc_tiling_on_sc=False),
    scratch_types=[
        pltpu.VMEM((HALF, CH), jnp.int32),
        pltpu.VMEM((HALF, CH), jnp.int32),
        pltpu.VMEM((NBUF, CH, HD), jnp.float32),
        pltpu.VMEM((CH,), jnp.float32),
        pltpu.VMEM((RPT,), jnp.float32),
        pltpu.VMEM_SHARED((NP, HD), jnp.float32),
        pltpu.VMEM_SHARED((NP,), jnp.float32),
        pltpu.SemaphoreType.DMA,
        pltpu.SemaphoreType.DMA,
        pltpu.SemaphoreType.DMA,
        pltpu.SemaphoreType.DMA,
        pltpu.SemaphoreType.DMA,
        pltpu.SemaphoreType.DMA,
        pltpu.SemaphoreType.DMA,
        pltpu.SemaphoreType.DMA,
        pltpu.SemaphoreType.DMA,
    ],
)
def _aggregate_kernel(h2_hbm, src_hbm, dst_hbm, ones_hbm, zrows_hbm, zcol_hbm,
                      agg_hbm, degi_hbm,
                      idxs_v, idxd_v, buf_v, ones_v, zb_v, agg_sh, degi_sh,
                      *sems):
    gsems = sems[:NBUF]
    osem = sems[NBUF]
    c = lax.axis_index("c")
    s = lax.axis_index("s")
    r0 = s * RPT
    pltpu.sync_copy(ones_hbm, ones_v)
    for j, rows in enumerate(_ROW_CHUNKS):
        pltpu.sync_copy(zrows_hbm.at[pl.ds(r0 + 128 * j, rows)],
                        buf_v.at[0, pl.ds(0, rows)])
        pltpu.sync_copy(buf_v.at[0, pl.ds(0, rows)],
                        agg_sh.at[pl.ds(r0 + 128 * j, rows)])
    pltpu.sync_copy(zcol_hbm.at[pl.ds(r0, RPT)], zb_v)
    pltpu.sync_copy(zb_v, degi_sh.at[pl.ds(r0, RPT)])
    plsc.subcore_barrier()

    def gstart(g, slot):
        pltpu.async_copy(h2_hbm.at[c].at[idxs_v.at[g]], buf_v.at[slot],
                         gsems[slot])

    def gwait(slot):
        pltpu.make_async_copy(h2_hbm.at[c].at[idxs_v.at[0]], buf_v.at[slot],
                              gsems[slot]).wait()

    for p in range(2):
        pltpu.sync_copy(src_hbm.at[s, pl.ds(p * HALF, HALF)], idxs_v)
        pltpu.sync_copy(dst_hbm.at[s, pl.ds(p * HALF, HALF)], idxd_v)
        count_here = jnp.equal(c, p)

        def scat(g, slot):
            pltpu.sync_copy(buf_v.at[slot], agg_sh.at[idxd_v.at[g]], add=True)

            @pl.when(count_here)
            def _():
                pltpu.async_copy(ones_v, degi_sh.at[idxd_v.at[g]], osem,
                                 add=True)

        for b in range(NBUF):
            gstart(b, b)

        def body(i, carry):
            g0 = NBUF * i
            for b in range(NBUF):
                gwait(b)
                scat(g0 + b, b)
                gstart(g0 + b + NBUF, b)
            return carry

        lax.fori_loop(0, (HALF - NBUF) // NBUF, body, 0)
        for b in range(NBUF):
            gwait(b)
            scat(HALF - NBUF + b, b)

        @pl.when(count_here)
        def _():
            def drain(i, carry):
                pltpu.make_async_copy(ones_v, degi_sh.at[idxd_v.at[0]],
                                      osem).wait()
                return carry
            lax.fori_loop(0, HALF, drain, 0)

    plsc.subcore_barrier()
    for j, rows in enumerate(_ROW_CHUNKS):
        pltpu.sync_copy(agg_sh.at[pl.ds(r0 + 128 * j, rows)],
                        buf_v.at[0, pl.ds(0, rows)])
        pltpu.sync_copy(buf_v.at[0, pl.ds(0, rows)],
                        agg_hbm.at[c, pl.ds(r0 + 128 * j, rows)])
    pltpu.sync_copy(degi_sh.at[pl.ds(r0, RPT)], zb_v)
    pltpu.sync_copy(zb_v, degi_hbm.at[pl.ds(c * NP + r0, RPT)])


def _scale_body(x_ref, deg_ref, h2_ref):
    i = pl.program_id(0)
    d = deg_ref[0] + deg_ref[1]
    h = x_ref[...] * lax.rsqrt(jnp.maximum(d, 1.0))
    rows = i * BLK + lax.broadcasted_iota(jnp.int32, (BLK, 1), 0)
    h = jnp.where(rows < N, h, 0.0)
    h2_ref[0] = h[:, :HD]
    h2_ref[1] = h[:, HD:]


def _scale_rows(x, degp):
    return pl.pallas_call(
        _scale_body,
        grid=(NP // BLK,),
        in_specs=[
            pl.BlockSpec((BLK, D), lambda i: (i, 0)),
            pl.BlockSpec((2, BLK, 1), lambda i: (0, i, 0)),
        ],
        out_specs=pl.BlockSpec((2, BLK, HD), lambda i: (0, i, 0)),
        out_shape=jax.ShapeDtypeStruct((2, NP, HD), jnp.float32),
    )(x, degp)


def _finish_body(agg_ref, deg_ref, w_ref, b_ref, y_ref):
    d = deg_ref[0] + deg_ref[1]
    n = lax.rsqrt(jnp.maximum(d, 1.0))
    w = w_ref[...]
    y_ref[...] = (jnp.dot(agg_ref[0] * n, w[:HD, :],
                          preferred_element_type=jnp.float32)
                  + jnp.dot(agg_ref[1] * n, w[HD:, :],
                            preferred_element_type=jnp.float32)
                  + b_ref[...])


def _finish(aggp, degp, W, b2):
    return pl.pallas_call(
        _finish_body,
        grid=(N // FBLK,),
        in_specs=[
            pl.BlockSpec((2, FBLK, HD), lambda i: (0, i, 0)),
            pl.BlockSpec((2, FBLK, 1), lambda i: (0, i, 0)),
            pl.BlockSpec((D, D), lambda i: (0, 0)),
            pl.BlockSpec((1, D), lambda i: (0, 0)),
        ],
        out_specs=pl.BlockSpec((FBLK, D), lambda i: (i, 0)),
        out_shape=jax.ShapeDtypeStruct((N, D), jnp.float32),
    )(aggp, degp, W, b2)


def kernel(x, edge_index, W, b):
    src = edge_index[0]
    dst = edge_index[1]

    pad = (N + jnp.arange(NT, dtype=jnp.int32))[:, None]
    pad = jnp.broadcast_to(pad, (NT, PADT))
    src3 = jnp.concatenate([src.reshape(NT, ECT), pad], 1).reshape(NT, NCHT, CH)
    dst3 = jnp.concatenate([dst.reshape(NT, ECT), pad], 1).reshape(NT, NCHT, CH)

    ones_col = jnp.ones((CH,), jnp.float32)
    zcol = jnp.zeros((NP,), jnp.float32)
    zrows = jnp.zeros((NP, HD), jnp.float32)

    deg_out_f = _degree_kernel(src3, ones_col, zcol)
    h2 = _scale_rows(x, deg_out_f.reshape(2, NP, 1))
    aggp, deg_in_f = _aggregate_kernel(h2, src3, dst3, ones_col, zrows, zcol)
    return _finish(aggp, deg_in_f.reshape(2, NP, 1), W, b.reshape(1, D))


# speedup vs baseline: 1.0735x; 1.0638x over previous
import functools

import jax
import jax.numpy as jnp
from jax import lax
from jax.experimental import pallas as pl
from jax.experimental.pallas import tpu as pltpu
from jax.experimental.pallas import tpu_sc as plsc

N = 10000
E = 320000
D = 128
NP = 10112
NW = 32
EPW = E // NW
CH = 128
NCH = 80
PADW = NCH * CH - EPW
RPT = NP // 16
_ROW_CHUNKS = (128, 128, 128, 128, 120)
BLK = NP // 4
FBLK = 2000

_mesh = plsc.VectorSubcoreMesh(core_axis_name="c", subcore_axis_name="s")


@functools.partial(
    pl.kernel,
    out_type=jax.ShapeDtypeStruct((2 * NP,), jnp.float32),
    mesh=_mesh,
    scratch_types=[
        pltpu.VMEM((NCH, CH), jnp.int32),
        pltpu.VMEM((CH,), jnp.float32),
        pltpu.VMEM((RPT,), jnp.float32),
        pltpu.VMEM_SHARED((NP,), jnp.float32),
    ],
)
def _degree_kernel(src_hbm, ones_hbm, zcol_hbm, dego_hbm,
                   idxs_v, ones_v, zb_v, dego_sh):
    c = lax.axis_index("c")
    s = lax.axis_index("s")
    wid = c * 16 + s
    r0 = s * RPT
    pltpu.sync_copy(src_hbm.at[wid], idxs_v)
    pltpu.sync_copy(ones_hbm, ones_v)
    pltpu.sync_copy(zcol_hbm.at[pl.ds(r0, RPT)], zb_v)
    pltpu.sync_copy(zb_v, dego_sh.at[pl.ds(r0, RPT)])
    plsc.subcore_barrier()

    def body(g, carry):
        pltpu.sync_copy(ones_v, dego_sh.at[idxs_v.at[g]], add=True)
        return carry

    lax.fori_loop(0, NCH, body, 0)
    plsc.subcore_barrier()
    pltpu.sync_copy(dego_sh.at[pl.ds(r0, RPT)], zb_v)
    pltpu.sync_copy(zb_v, dego_hbm.at[pl.ds(c * NP + r0, RPT)])


@functools.partial(
    pl.kernel,
    out_type=(
        jax.ShapeDtypeStruct((2, NP, D), jnp.float32),
        jax.ShapeDtypeStruct((2 * NP,), jnp.float32),
    ),
    mesh=_mesh,
    scratch_types=[
        pltpu.VMEM((NCH // 2, CH), jnp.int32),
        pltpu.VMEM((NCH // 2, CH), jnp.int32),
        pltpu.VMEM((2, CH, D), jnp.float32),
        pltpu.VMEM((CH,), jnp.float32),
        pltpu.VMEM((RPT,), jnp.float32),
        pltpu.VMEM_SHARED((NP, D), jnp.float32),
        pltpu.VMEM_SHARED((NP,), jnp.float32),
        pltpu.SemaphoreType.DMA,
        pltpu.SemaphoreType.DMA,
        pltpu.SemaphoreType.DMA,
        pltpu.SemaphoreType.DMA,
        pltpu.SemaphoreType.DMA,
    ],
)
def _aggregate_kernel(h_hbm, src_hbm, dst_hbm, ones_hbm, zrows_hbm, zcol_hbm,
                      agg_hbm, degi_hbm,
                      idxs_v, idxd_v, buf_v, ones_v, zb_v, agg_sh, degi_sh,
                      sem0, sem1, sem2, sem3, osem):
    c = lax.axis_index("c")
    s = lax.axis_index("s")
    wid = c * 16 + s
    r0 = s * RPT
    pltpu.sync_copy(ones_hbm, ones_v)
    for j, rows in enumerate(_ROW_CHUNKS):
        pltpu.sync_copy(zrows_hbm.at[pl.ds(r0 + 128 * j, rows)],
                        buf_v.at[0, pl.ds(0, rows)])
        pltpu.sync_copy(buf_v.at[0, pl.ds(0, rows)],
                        agg_sh.at[pl.ds(r0 + 128 * j, rows)])
    pltpu.sync_copy(zcol_hbm.at[pl.ds(r0, RPT)], zb_v)
    pltpu.sync_copy(zb_v, degi_sh.at[pl.ds(r0, RPT)])
    plsc.subcore_barrier()

    gsems = (sem0, sem1)
    ssems = (sem2, sem3)
    half = NCH // 2

    def gstart(g, slot):
        pltpu.async_copy(h_hbm.at[idxs_v.at[g]], buf_v.at[slot], gsems[slot])

    def gwait(slot):
        pltpu.make_async_copy(h_hbm.at[idxs_v.at[0]], buf_v.at[slot],
                              gsems[slot]).wait()

    def sstart(g, slot):
        pltpu.async_copy(buf_v.at[slot], agg_sh.at[idxd_v.at[g]], ssems[slot],
                         add=True)

    def swait(slot):
        pltpu.make_async_copy(buf_v.at[slot], agg_sh.at[idxd_v.at[0]],
                              ssems[slot]).wait()

    def ostart(g):
        pltpu.async_copy(ones_v, degi_sh.at[idxd_v.at[g]], osem, add=True)

    for p in range(2):
        pltpu.sync_copy(src_hbm.at[wid, pl.ds(p * half, half)], idxs_v)
        pltpu.sync_copy(dst_hbm.at[wid, pl.ds(p * half, half)], idxd_v)
        gstart(0, 0)
        gstart(1, 1)

        def body(i, carry):
            g = 2 * i
            gwait(0)
            sstart(g, 0)
            ostart(g)
            swait(0)
            gstart(g + 2, 0)
            gwait(1)
            sstart(g + 1, 1)
            ostart(g + 1)
            swait(1)
            gstart(g + 3, 1)
            return carry

        lax.fori_loop(0, (half - 2) // 2, body, 0)
        gwait(0)
        sstart(half - 2, 0)
        ostart(half - 2)
        swait(0)
        gwait(1)
        sstart(half - 1, 1)
        ostart(half - 1)
        swait(1)

        def odrain(i, carry):
            pltpu.make_async_copy(ones_v, degi_sh.at[idxd_v.at[0]],
                                  osem).wait()
            return carry

        lax.fori_loop(0, half, odrain, 0)

    plsc.subcore_barrier()
    for j, rows in enumerate(_ROW_CHUNKS):
        pltpu.sync_copy(agg_sh.at[pl.ds(r0 + 128 * j, rows)],
                        buf_v.at[0, pl.ds(0, rows)])
        pltpu.sync_copy(buf_v.at[0, pl.ds(0, rows)],
                        agg_hbm.at[c, pl.ds(r0 + 128 * j, rows)])
    pltpu.sync_copy(degi_sh.at[pl.ds(r0, RPT)], zb_v)
    pltpu.sync_copy(zb_v, degi_hbm.at[pl.ds(c * NP + r0, RPT)])


def _scale_body(x_ref, deg_ref, h_ref):
    d = deg_ref[0] + deg_ref[1]
    h_ref[...] = x_ref[...] * lax.rsqrt(jnp.maximum(d, 1.0))


def _scale_rows(x_pad, degp):
    return pl.pallas_call(
        _scale_body,
        grid=(NP // BLK,),
        in_specs=[
            pl.BlockSpec((BLK, D), lambda i: (i, 0)),
            pl.BlockSpec((2, BLK, 1), lambda i: (0, i, 0)),
        ],
        out_specs=pl.BlockSpec((BLK, D), lambda i: (i, 0)),
        out_shape=jax.ShapeDtypeStruct((NP, D), jnp.float32),
    )(x_pad, degp)


def _finish_body(agg_ref, deg_ref, w_ref, b_ref, y_ref):
    a = agg_ref[0] + agg_ref[1]
    d = deg_ref[0] + deg_ref[1]
    a = a * lax.rsqrt(jnp.maximum(d, 1.0))
    y_ref[...] = (jnp.dot(a, w_ref[...], preferred_element_type=jnp.float32)
                  + b_ref[...])


def _finish(aggp, degp, W, b2):
    return pl.pallas_call(
        _finish_body,
        grid=(N // FBLK,),
        in_specs=[
            pl.BlockSpec((2, FBLK, D), lambda i: (0, i, 0)),
            pl.BlockSpec((2, FBLK, 1), lambda i: (0, i, 0)),
            pl.BlockSpec((D, D), lambda i: (0, 0)),
            pl.BlockSpec((1, D), lambda i: (0, 0)),
        ],
        out_specs=pl.BlockSpec((FBLK, D), lambda i: (i, 0)),
        out_shape=jax.ShapeDtypeStruct((N, D), jnp.float32),
    )(aggp, degp, W, b2)


def kernel(x, edge_index, W, b):
    src = edge_index[0]
    dst = edge_index[1]

    pad = (N + (jnp.arange(NW, dtype=jnp.int32) % 16))[:, None]
    pad = jnp.broadcast_to(pad, (NW, PADW))
    srcp = jnp.concatenate([src.reshape(NW, EPW), pad], 1).reshape(NW, NCH, CH)
    dstp = jnp.concatenate([dst.reshape(NW, EPW), pad], 1).reshape(NW, NCH, CH)

    x_pad = jnp.concatenate([x, jnp.zeros((NP - N, D), jnp.float32)], 0)
    ones_col = jnp.ones((CH,), jnp.float32)
    zcol = jnp.zeros((NP,), jnp.float32)
    zrows = jnp.zeros((NP, D), jnp.float32)

    deg_out_f = _degree_kernel(srcp, ones_col, zcol)
    h = _scale_rows(x_pad, deg_out_f.reshape(2, NP, 1))
    aggp, deg_in_f = _aggregate_kernel(h, srcp, dstp, ones_col, zrows, zcol)
    return _finish(aggp, deg_in_f.reshape(2, NP, 1), W, b.reshape(1, D))


# speedup vs baseline: 1.1039x; 1.0283x over previous
import functools

import jax
import jax.numpy as jnp
from jax import lax
from jax.experimental import pallas as pl
from jax.experimental.pallas import tpu as pltpu
from jax.experimental.pallas import tpu_sc as plsc

N = 10000
E = 320000
D = 128
NP = 10112
NW = 32
EPW = E // NW
CH = 128
NCH = 80
PADW = NCH * CH - EPW
RPT = NP // 16
_ROW_CHUNKS = (128, 128, 128, 128, 120)
BLK = NP // 4
FBLK = 2000

_mesh = plsc.VectorSubcoreMesh(core_axis_name="c", subcore_axis_name="s")


@functools.partial(
    pl.kernel,
    out_type=(
        jax.ShapeDtypeStruct((2 * NP,), jnp.float32),
        jax.ShapeDtypeStruct((2 * NP,), jnp.float32),
    ),
    mesh=_mesh,
    scratch_types=[
        pltpu.VMEM((NCH, CH), jnp.int32),
        pltpu.VMEM((NCH, CH), jnp.int32),
        pltpu.VMEM((CH,), jnp.float32),
        pltpu.VMEM((RPT,), jnp.float32),
        pltpu.VMEM_SHARED((NP,), jnp.float32),
        pltpu.VMEM_SHARED((NP,), jnp.float32),
    ],
)
def _degree_kernel(src_hbm, dst_hbm, ones_hbm, zcol_hbm, dego_hbm, degi_hbm,
                   idxs_v, idxd_v, ones_v, zb_v, dego_sh, degi_sh):
    c = lax.axis_index("c")
    s = lax.axis_index("s")
    wid = c * 16 + s
    r0 = s * RPT
    pltpu.sync_copy(src_hbm.at[wid], idxs_v)
    pltpu.sync_copy(dst_hbm.at[wid], idxd_v)
    pltpu.sync_copy(ones_hbm, ones_v)
    pltpu.sync_copy(zcol_hbm.at[pl.ds(r0, RPT)], zb_v)
    pltpu.sync_copy(zb_v, dego_sh.at[pl.ds(r0, RPT)])
    pltpu.sync_copy(zb_v, degi_sh.at[pl.ds(r0, RPT)])
    plsc.subcore_barrier()

    def body(g, carry):
        pltpu.sync_copy(ones_v, dego_sh.at[idxs_v.at[g]], add=True)
        pltpu.sync_copy(ones_v, degi_sh.at[idxd_v.at[g]], add=True)
        return carry

    lax.fori_loop(0, NCH, body, 0)
    plsc.subcore_barrier()
    pltpu.sync_copy(dego_sh.at[pl.ds(r0, RPT)], zb_v)
    pltpu.sync_copy(zb_v, dego_hbm.at[pl.ds(c * NP + r0, RPT)])
    pltpu.sync_copy(degi_sh.at[pl.ds(r0, RPT)], zb_v)
    pltpu.sync_copy(zb_v, degi_hbm.at[pl.ds(c * NP + r0, RPT)])


@functools.partial(
    pl.kernel,
    out_type=jax.ShapeDtypeStruct((2, NP, D), jnp.float32),
    mesh=_mesh,
    scratch_types=[
        pltpu.VMEM((NCH // 2, CH), jnp.int32),
        pltpu.VMEM((NCH // 2, CH), jnp.int32),
        pltpu.VMEM((2, CH, D), jnp.float32),
        pltpu.VMEM_SHARED((NP, D), jnp.float32),
        pltpu.SemaphoreType.DMA,
        pltpu.SemaphoreType.DMA,
        pltpu.SemaphoreType.DMA,
        pltpu.SemaphoreType.DMA,
    ],
)
def _aggregate_kernel(h_hbm, src_hbm, dst_hbm, zrows_hbm, agg_hbm,
                      idxs_v, idxd_v, buf_v, agg_sh, sem0, sem1, sem2, sem3):
    c = lax.axis_index("c")
    s = lax.axis_index("s")
    wid = c * 16 + s
    r0 = s * RPT
    for j, rows in enumerate(_ROW_CHUNKS):
        pltpu.sync_copy(zrows_hbm.at[pl.ds(r0 + 128 * j, rows)],
                        buf_v.at[0, pl.ds(0, rows)])
        pltpu.sync_copy(buf_v.at[0, pl.ds(0, rows)],
                        agg_sh.at[pl.ds(r0 + 128 * j, rows)])
    plsc.subcore_barrier()

    gsems = (sem0, sem1)
    ssems = (sem2, sem3)
    half = NCH // 2

    def gstart(g, slot):
        pltpu.async_copy(h_hbm.at[idxs_v.at[g]], buf_v.at[slot], gsems[slot])

    def gwait(slot):
        pltpu.make_async_copy(h_hbm.at[idxs_v.at[0]], buf_v.at[slot],
                              gsems[slot]).wait()

    def sstart(g, slot):
        pltpu.async_copy(buf_v.at[slot], agg_sh.at[idxd_v.at[g]], ssems[slot],
                         add=True)

    def swait(slot):
        pltpu.make_async_copy(buf_v.at[slot], agg_sh.at[idxd_v.at[0]],
                              ssems[slot]).wait()

    for p in range(2):
        pltpu.sync_copy(src_hbm.at[wid, pl.ds(p * half, half)], idxs_v)
        pltpu.sync_copy(dst_hbm.at[wid, pl.ds(p * half, half)], idxd_v)
        gstart(0, 0)
        gstart(1, 1)

        def body(i, carry):
            g = 2 * i
            gwait(0)
            sstart(g, 0)
            swait(0)
            gstart(g + 2, 0)
            gwait(1)
            sstart(g + 1, 1)
            swait(1)
            gstart(g + 3, 1)
            return carry

        lax.fori_loop(0, (half - 2) // 2, body, 0)
        gwait(0)
        sstart(half - 2, 0)
        swait(0)
        gwait(1)
        sstart(half - 1, 1)
        swait(1)

    plsc.subcore_barrier()
    for j, rows in enumerate(_ROW_CHUNKS):
        pltpu.sync_copy(agg_sh.at[pl.ds(r0 + 128 * j, rows)],
                        buf_v.at[0, pl.ds(0, rows)])
        pltpu.sync_copy(buf_v.at[0, pl.ds(0, rows)],
                        agg_hbm.at[c, pl.ds(r0 + 128 * j, rows)])


def _scale_body(x_ref, deg_ref, h_ref):
    d = deg_ref[0] + deg_ref[1]
    h_ref[...] = x_ref[...] * lax.rsqrt(jnp.maximum(d, 1.0))


def _scale_rows(x_pad, degp):
    return pl.pallas_call(
        _scale_body,
        grid=(NP // BLK,),
        in_specs=[
            pl.BlockSpec((BLK, D), lambda i: (i, 0)),
            pl.BlockSpec((2, BLK, 1), lambda i: (0, i, 0)),
        ],
        out_specs=pl.BlockSpec((BLK, D), lambda i: (i, 0)),
        out_shape=jax.ShapeDtypeStruct((NP, D), jnp.float32),
    )(x_pad, degp)


def _finish_body(agg_ref, deg_ref, w_ref, b_ref, y_ref):
    a = agg_ref[0] + agg_ref[1]
    d = deg_ref[0] + deg_ref[1]
    a = a * lax.rsqrt(jnp.maximum(d, 1.0))
    y_ref[...] = (jnp.dot(a, w_ref[...], preferred_element_type=jnp.float32)
                  + b_ref[...])


def _finish(aggp, degp, W, b2):
    return pl.pallas_call(
        _finish_body,
        grid=(N // FBLK,),
        in_specs=[
            pl.BlockSpec((2, FBLK, D), lambda i: (0, i, 0)),
            pl.BlockSpec((2, FBLK, 1), lambda i: (0, i, 0)),
            pl.BlockSpec((D, D), lambda i: (0, 0)),
            pl.BlockSpec((1, D), lambda i: (0, 0)),
        ],
        out_specs=pl.BlockSpec((FBLK, D), lambda i: (i, 0)),
        out_shape=jax.ShapeDtypeStruct((N, D), jnp.float32),
    )(aggp, degp, W, b2)


def kernel(x, edge_index, W, b):
    src = edge_index[0]
    dst = edge_index[1]

    pad = (N + (jnp.arange(NW, dtype=jnp.int32) % 16))[:, None]
    pad = jnp.broadcast_to(pad, (NW, PADW))
    srcp = jnp.concatenate([src.reshape(NW, EPW), pad], 1).reshape(NW, NCH, CH)
    dstp = jnp.concatenate([dst.reshape(NW, EPW), pad], 1).reshape(NW, NCH, CH)

    x_pad = jnp.concatenate([x, jnp.zeros((NP - N, D), jnp.float32)], 0)
    ones_col = jnp.ones((CH,), jnp.float32)
    zcol = jnp.zeros((NP,), jnp.float32)
    zrows = jnp.zeros((NP, D), jnp.float32)

    deg_out_f, deg_in_f = _degree_kernel(srcp, dstp, ones_col, zcol)
    h = _scale_rows(x_pad, deg_out_f.reshape(2, NP, 1))
    aggp = _aggregate_kernel(h, srcp, dstp, zrows)
    return _finish(aggp, deg_in_f.reshape(2, NP, 1), W, b.reshape(1, D))
